# baseline (speedup 1.0000x reference)
import functools

import jax
import jax.numpy as jnp
from jax import lax
from jax.experimental import pallas as pl
from jax.experimental.pallas import tpu as pltpu
from jax.experimental.pallas import tpu_sc as plsc

N = 16384
D = 20
THRESHOLDS = (1, 2, 3, 4, 8, 16, 32, 64)

NC = 2
NS = 16
NW = NC * NS
B_PER_W = N // NW
L = 16
NBLK = B_PER_W // L
NCHUNK = 4
CHUNK = B_PER_W // NCHUNK
BLK_PER_CHUNK = NBLK // NCHUNK


def _sc_body(lengths_hbm, table_hbm, out_hbm, len_v, table_v, rows_v, sem):
    wid = lax.axis_index("s") * NC + lax.axis_index("c")
    base = wid * B_PER_W

    cp_tab = pltpu.async_copy(table_hbm, table_v, sem)
    pltpu.sync_copy(lengths_hbm.at[pl.ds(base, B_PER_W)], len_v)
    cp_tab.wait()

    lanes = lax.iota(jnp.int32, L)
    cols = [jnp.full((L,), j, jnp.int32) for j in range(D)]

    @plsc.parallel_loop(0, NBLK // 2, unroll=4)
    def blk_lo(b):
        lv = len_v[pl.ds(b * L, L)]
        cs = [(lv >= t).astype(jnp.int32) for t in THRESHOLDS]
        while len(cs) > 1:
            cs = [a + b for a, b in zip(cs[::2], cs[1::2])]
        acc = cs[0]
        rows = lanes + b * L

        @plsc.parallel_loop(0, D, unroll=4)
        def colloop(j):
            col = jnp.full((L,), j, jnp.int32)
            vals = plsc.load_gather(table_v, [acc, col])
            plsc.store_scatter(rows_v, [rows, col], vals)

    half = B_PER_W // 2
    cp_lo = pltpu.async_copy(
        rows_v.at[pl.ds(0, half)], out_hbm.at[pl.ds(base, half)], sem
    )

    @plsc.parallel_loop(NBLK // 2, NBLK, unroll=4)
    def blk_hi(b):
        lv = len_v[pl.ds(b * L, L)]
        cs = [(lv >= t).astype(jnp.int32) for t in THRESHOLDS]
        while len(cs) > 1:
            cs = [a + b for a, b in zip(cs[::2], cs[1::2])]
        acc = cs[0]
        rows = lanes + b * L

        @plsc.parallel_loop(0, D, unroll=4)
        def colloop(j):
            col = jnp.full((L,), j, jnp.int32)
            vals = plsc.load_gather(table_v, [acc, col])
            plsc.store_scatter(rows_v, [rows, col], vals)

    pltpu.sync_copy(
        rows_v.at[pl.ds(half, half)], out_hbm.at[pl.ds(base + half, half)]
    )
    cp_lo.wait()


_embed = functools.partial(
    pl.kernel,
    mesh=plsc.VectorSubcoreMesh(core_axis_name="c", subcore_axis_name="s"),
    compiler_params=pltpu.CompilerParams(needs_layout_passes=False),
    out_type=jax.ShapeDtypeStruct((N, D), jnp.float32),
    scratch_types=[
        pltpu.VMEM((B_PER_W,), jnp.int32),
        pltpu.VMEM((9, D), jnp.float32),
        pltpu.VMEM((B_PER_W, D), jnp.float32),
        pltpu.SemaphoreType.DMA,
    ],
)(_sc_body)


def kernel(lengths, table):
    return _embed(lengths.astype(jnp.int32), table.astype(jnp.float32))


# speedup vs baseline: 1.0097x; 1.0097x over previous
import functools

import jax
import jax.numpy as jnp
from jax import lax
from jax.experimental import pallas as pl
from jax.experimental.pallas import tpu as pltpu
from jax.experimental.pallas import tpu_sc as plsc

N = 16384
D = 20
THRESHOLDS = (1, 2, 3, 4, 8, 16, 32, 64)

NC = 2
NS = 16
NW = NC * NS
B_PER_W = N // NW
L = 16
NBLK = B_PER_W // L
NCHUNK = 4
CHUNK = B_PER_W // NCHUNK
BLK_PER_CHUNK = NBLK // NCHUNK


def _sc_body(lengths_hbm, table_hbm, out_hbm, len_v, table_v, rows_v, sem):
    wid = lax.axis_index("s") * NC + lax.axis_index("c")
    base = wid * B_PER_W

    cp_tab = pltpu.async_copy(table_hbm, table_v, sem)
    pltpu.sync_copy(lengths_hbm.at[pl.ds(base, B_PER_W)], len_v)
    cp_tab.wait()

    lanes = lax.iota(jnp.int32, L)
    cols = [jnp.full((L,), j, jnp.int32) for j in range(D)]

    @plsc.parallel_loop(0, NBLK // 2, unroll=2)
    def blk_lo(b):
        lv = len_v[pl.ds(b * L, L)]
        acc = jnp.zeros((L,), jnp.int32)
        for t in THRESHOLDS:
            acc = acc + (lv >= t).astype(jnp.int32)
        rows = lanes + b * L
        for j in range(D):
            vals = plsc.load_gather(table_v, [acc, cols[j]])
            plsc.store_scatter(rows_v, [rows, cols[j]], vals)

    half = B_PER_W // 2
    cp_lo = pltpu.async_copy(
        rows_v.at[pl.ds(0, half)], out_hbm.at[pl.ds(base, half)], sem
    )

    @plsc.parallel_loop(NBLK // 2, NBLK, unroll=2)
    def blk_hi(b):
        lv = len_v[pl.ds(b * L, L)]
        acc = jnp.zeros((L,), jnp.int32)
        for t in THRESHOLDS:
            acc = acc + (lv >= t).astype(jnp.int32)
        rows = lanes + b * L
        for j in range(D):
            vals = plsc.load_gather(table_v, [acc, cols[j]])
            plsc.store_scatter(rows_v, [rows, cols[j]], vals)

    pltpu.sync_copy(
        rows_v.at[pl.ds(half, half)], out_hbm.at[pl.ds(base + half, half)]
    )
    cp_lo.wait()


_embed = functools.partial(
    pl.kernel,
    mesh=plsc.VectorSubcoreMesh(core_axis_name="c", subcore_axis_name="s"),
    compiler_params=pltpu.CompilerParams(needs_layout_passes=False),
    out_type=jax.ShapeDtypeStruct((N, D), jnp.float32),
    scratch_types=[
        pltpu.VMEM((B_PER_W,), jnp.int32),
        pltpu.VMEM((9, D), jnp.float32),
        pltpu.VMEM((B_PER_W, D), jnp.float32),
        pltpu.SemaphoreType.DMA,
    ],
)(_sc_body)


def kernel(lengths, table):
    return _embed(lengths.astype(jnp.int32), table.astype(jnp.float32))


# speedup vs baseline: 1.0137x; 1.0039x over previous
import functools

import jax
import jax.numpy as jnp
from jax import lax
from jax.experimental import pallas as pl
from jax.experimental.pallas import tpu as pltpu
from jax.experimental.pallas import tpu_sc as plsc

N = 16384
D = 20
THRESHOLDS = (1, 2, 3, 4, 8, 16, 32, 64)

NC = 2
NS = 16
NW = NC * NS
B_PER_W = N // NW
L = 16
NBLK = B_PER_W // L


def _sc_body(lengths_hbm, table_hbm, out_hbm, len_v, table_v, rows_v, sem):
    wid = lax.axis_index("s") * NC + lax.axis_index("c")
    base = wid * B_PER_W

    cp_tab = pltpu.async_copy(table_hbm, table_v, sem)
    pltpu.sync_copy(lengths_hbm.at[pl.ds(base, B_PER_W)], len_v)
    cp_tab.wait()

    lanes = lax.iota(jnp.int32, L)
    cols = [jnp.full((L,), j, jnp.int32) for j in range(D)]

    def lookup_blocks(lo, hi):
        @plsc.parallel_loop(lo, hi, unroll=2)
        def blk(b):
            lv = len_v[pl.ds(b * L, L)]
            acc = jnp.zeros((L,), jnp.int32)
            for t in THRESHOLDS:
                acc = acc + (lv >= t).astype(jnp.int32)
            rows = lanes + b * L
            for j in range(D):
                vals = plsc.load_gather(table_v, [acc, cols[j]])
                plsc.store_scatter(rows_v, [rows, cols[j]], vals)

    half = B_PER_W // 2
    lookup_blocks(0, NBLK // 2)
    cp_lo = pltpu.async_copy(
        rows_v.at[pl.ds(0, half)], out_hbm.at[pl.ds(base, half)], sem
    )
    lookup_blocks(NBLK // 2, NBLK)
    pltpu.sync_copy(
        rows_v.at[pl.ds(half, half)], out_hbm.at[pl.ds(base + half, half)]
    )
    cp_lo.wait()


_embed = functools.partial(
    pl.kernel,
    mesh=plsc.VectorSubcoreMesh(core_axis_name="c", subcore_axis_name="s"),
    compiler_params=pltpu.CompilerParams(needs_layout_passes=False),
    out_type=jax.ShapeDtypeStruct((N, D), jnp.float32),
    scratch_types=[
        pltpu.VMEM((B_PER_W,), jnp.int32),
        pltpu.VMEM((9, D), jnp.float32),
        pltpu.VMEM((B_PER_W, D), jnp.float32),
        pltpu.SemaphoreType.DMA,
    ],
)(_sc_body)


def kernel(lengths, table):
    return _embed(lengths.astype(jnp.int32), table.astype(jnp.float32))
